# baseline (speedup 1.0000x reference)
import functools

import jax
import jax.numpy as jnp
from jax import lax
from jax.experimental import pallas as pl
from jax.experimental.pallas import tpu as pltpu
from jax.experimental.pallas import tpu_sc as plsc

D = 64
L = 16
IDXW = 128
CHUNK = 256
NW = 32


def _lookup(phys2d, colb1d, w128):
    nidx_rows = phys2d.shape[0]
    b_total = nidx_rows * IDXW
    per_w = b_total // NW
    nch = per_w // CHUNK
    g_per_chunk = CHUNK // IDXW
    idx_rows_per_w = per_w // IDXW
    owords_chunk = CHUNK * (D // 2)
    owords_w = per_w * (D // 2)

    mesh = plsc.VectorSubcoreMesh(core_axis_name="c", subcore_axis_name="s")

    @functools.partial(
        pl.kernel,
        out_type=jax.ShapeDtypeStruct((b_total * (D // 2),), jnp.int32),
        mesh=mesh,
        scratch_types=[
            pltpu.VMEM((g_per_chunk, IDXW), jnp.int32),
            pltpu.VMEM((CHUNK,), jnp.int32),
            pltpu.VMEM((CHUNK, 2 * D), jnp.float32),
            pltpu.VMEM((owords_chunk,), jnp.int32),
            pltpu.SemaphoreType.DMA,
        ],
        compiler_params=pltpu.CompilerParams(
            needs_layout_passes=False, use_tc_tiling_on_sc=False
        ),
    )
    def run(phys_hbm, colb_hbm, tbl_hbm, out_hbm, phys_v, colb_s, rows_v,
            out_v, sem):
        cid = lax.axis_index("c")
        sid = lax.axis_index("s")
        wid = sid * 2 + cid
        idx_row0 = wid * idx_rows_per_w
        flat0 = wid * per_w
        out0 = wid * owords_w

        iota = lax.iota(jnp.int32, L)

        def chunk_body(t, carry):
            pltpu.sync_copy(
                phys_hbm.at[pl.ds(idx_row0 + t * g_per_chunk, g_per_chunk)],
                phys_v,
            )
            pltpu.sync_copy(
                colb_hbm.at[pl.ds(flat0 + t * CHUNK, CHUNK)], colb_s
            )
            cps = []
            for g in range(g_per_chunk):
                cps.append(
                    pltpu.async_copy(
                        tbl_hbm.at[phys_v.at[g]],
                        rows_v.at[pl.ds(g * IDXW, IDXW)],
                        sem,
                    )
                )
            for cp in cps:
                cp.wait()

            def cast_block(jb, c2):
                j0 = jb * L
                cbv = colb_s[pl.ds(j0, L)]
                for k in range(L):
                    j = j0 + k
                    cb = cbv[k]
                    jv = jnp.full((L,), j, jnp.int32)
                    for h in range(D // 32):
                        ev = plsc.load_gather(
                            rows_v, [jv, cb + h * 32 + 2 * iota]
                        )
                        od = plsc.load_gather(
                            rows_v, [jv, cb + h * 32 + 2 * iota + 1]
                        )
                        p = plsc.pack(
                            ev, od, format=plsc.PackFormat.INTERLEAVED
                        )
                        w = plsc.bitcast(p, jnp.int32)
                        out_v[pl.ds(j * (D // 2) + h * L, L)] = w
                return c2

            lax.fori_loop(0, CHUNK // L, cast_block, 0)
            pltpu.sync_copy(
                out_v,
                out_hbm.at[pl.ds(out0 + t * owords_chunk, owords_chunk)],
            )
            return carry

        lax.fori_loop(0, nch, chunk_body, 0)

    return run(phys2d, colb1d, w128)


def kernel(input_ids, weight):
    b, s = input_ids.shape
    ids = input_ids.reshape(-1).astype(jnp.int32)
    phys2d = (ids >> 1).reshape(-1, IDXW)
    colb1d = (ids & 1) * D
    w128 = weight.reshape(-1, 2 * D)
    owords = _lookup(phys2d, colb1d, w128)
    out = jax.lax.bitcast_convert_type(owords, jnp.bfloat16)
    return out.reshape(b, s, D)
